# baseline (speedup 1.0000x reference)
import functools

import jax
import jax.numpy as jnp
import numpy as np
from jax import lax
from jax.experimental import pallas as pl
from jax.experimental.pallas import tpu as pltpu
from jax.experimental.pallas import tpu_sc as plsc

MODEL_DIM = 2048
MAX_SEQ_LEN = 8192


def _pe_table_np(seq_len: int, model_dim: int) -> np.ndarray:
    positions = np.arange(0, seq_len, dtype=np.float32)[:, None]
    i = np.arange(0, model_dim // 2, dtype=np.float32)
    frequencies = np.power(np.float32(10000.0), 2.0 * i / np.float32(model_dim))
    pe = np.zeros((seq_len, model_dim), dtype=np.float32)
    pe[:, 0::2] = np.sin(positions / frequencies)
    pe[:, 1::2] = np.cos(positions / frequencies)
    return pe


_PE = _pe_table_np(MAX_SEQ_LEN, MODEL_DIM)


def _pack_pe_bf16(pe: np.ndarray) -> np.ndarray:
    u = pe.view(np.uint32)
    bits = ((u + 0x7FFF + ((u >> 16) & 1)) >> 16).astype(np.uint32)
    s, d = pe.shape
    b = bits.reshape(s, d // 32, 2, 16)
    packed = (b[:, :, 1, :] << 16) | b[:, :, 0, :]
    return packed.reshape(s, d // 2).view(np.int32)


_PE_PACKED = _pack_pe_bf16(_PE)

_NUM_WORKERS = 32
_P = 4
_NBUF = 3


def _make_sc_kernel(batch: int, seq_len: int, d: int):
    mesh = plsc.VectorSubcoreMesh(core_axis_name="c", subcore_axis_name="s")
    pos_per_w = seq_len // _NUM_WORKERS
    n_chunks = pos_per_w // _P
    n_rows = batch * seq_len
    dv = d // 16

    @functools.partial(
        pl.kernel,
        mesh=mesh,
        out_type=jax.ShapeDtypeStruct((n_rows, d), jnp.float32),
        scratch_types=[
            pltpu.VMEM((batch, pos_per_w), jnp.int32),
            pltpu.VMEM((_NBUF, batch, _P, d), jnp.float32),
            pltpu.VMEM((_NBUF, _P, d // 2), jnp.int32),
            pltpu.SemaphoreType.DMA,
            pltpu.SemaphoreType.DMA,
            pltpu.SemaphoreType.DMA,
            pltpu.SemaphoreType.DMA,
            pltpu.SemaphoreType.DMA,
            pltpu.SemaphoreType.DMA,
        ],
    )
    def body(ids_hbm, table_hbm, pe_hbm, out_hbm, idx_v, rows_v, pe_v,
             gsem0, gsem1, gsem2, ssem0, ssem1, ssem2):
        nc = 2
        wid = lax.axis_index("s") * nc + lax.axis_index("c")
        pos0 = wid * pos_per_w

        pltpu.sync_copy(ids_hbm.at[pl.ds(wid * batch, batch)], idx_v)

        gsems = (gsem0, gsem1, gsem2)
        ssems = (ssem0, ssem1, ssem2)

        def load_descs(c, buf):
            descs = [
                pltpu.make_async_copy(
                    table_hbm.at[idx_v.at[b, pl.ds(c * _P, _P)]],
                    rows_v.at[buf, b], gsems[buf])
                for b in range(batch)
            ]
            descs.append(pltpu.make_async_copy(
                pe_hbm.at[pl.ds(pos0 + c * _P, _P)], pe_v.at[buf], gsems[buf]))
            return descs

        def store_descs(c, buf):
            return [
                pltpu.make_async_copy(
                    rows_v.at[buf, b],
                    out_hbm.at[pl.ds(b * seq_len + pos0 + c * _P, _P)],
                    ssems[buf])
                for b in range(batch)
            ]

        def multiply(buf):
            hi_mask = jnp.int32(-65536)

            def mul_body(g, _):
                colp = g * 16
                col = g * 32
                for p in range(_P):
                    pv = pe_v[buf, p, pl.ds(colp, 16)]
                    pe_lo = lax.bitcast_convert_type(pv << 16, jnp.float32)
                    pe_hi = lax.bitcast_convert_type(pv & hi_mask, jnp.float32)
                    for b in range(batch):
                        rows_v[buf, b, p, pl.ds(col, 16)] = (
                            rows_v[buf, b, p, pl.ds(col, 16)] * pe_lo
                        )
                        rows_v[buf, b, p, pl.ds(col + 16, 16)] = (
                            rows_v[buf, b, p, pl.ds(col + 16, 16)] * pe_hi
                        )
                return 0

            lax.fori_loop(0, d // 32, mul_body, 0)

        for desc in load_descs(0, 0):
            desc.start()
        for desc in load_descs(1, 1):
            desc.start()

        def chunk_step(c, buf):
            ahead = (buf + 2) % _NBUF

            @pl.when(c + 2 < n_chunks)
            def _():
                @pl.when(c >= 1)
                def _():
                    for desc in store_descs(c - 1, ahead):
                        desc.wait()

                for desc in load_descs(c + 2, ahead):
                    desc.start()

            for desc in load_descs(c, buf):
                desc.wait()
            multiply(buf)
            for desc in store_descs(c, buf):
                desc.start()

        def group_body(g, _):
            for local in range(_NBUF):
                c = _NBUF * g + local
                chunk_step(c, local)
            return 0

        n_groups = n_chunks // _NBUF
        lax.fori_loop(0, n_groups, group_body, 0)
        for c in range(n_groups * _NBUF, n_chunks):
            chunk_step(c, c % _NBUF)

        for c in range(n_chunks - _NBUF, n_chunks):
            for desc in store_descs(c, c % _NBUF):
                desc.wait()

    return body


def kernel(input_ids, embedding_table):
    b, seq_len = input_ids.shape
    d = embedding_table.shape[1]
    pos_per_w = seq_len // _NUM_WORKERS
    ids_perm = (
        input_ids.astype(jnp.int32)
        .reshape(b, _NUM_WORKERS, pos_per_w)
        .transpose(1, 0, 2)
        .reshape(_NUM_WORKERS * b, pos_per_w)
    )
    pe = jnp.asarray(_PE_PACKED[:seq_len])
    out = _make_sc_kernel(b, seq_len, d)(ids_perm, embedding_table, pe)
    return out.reshape(b, seq_len, d)


# speedup vs baseline: 2.3842x; 2.3842x over previous
import functools

import jax
import jax.numpy as jnp
import numpy as np
from jax import lax
from jax.experimental import pallas as pl
from jax.experimental.pallas import tpu as pltpu
from jax.experimental.pallas import tpu_sc as plsc

MODEL_DIM = 2048
MAX_SEQ_LEN = 8192


def _pe_table_np(seq_len: int, model_dim: int) -> np.ndarray:
    positions = np.arange(0, seq_len, dtype=np.float32)[:, None]
    i = np.arange(0, model_dim // 2, dtype=np.float32)
    frequencies = np.power(np.float32(10000.0), 2.0 * i / np.float32(model_dim))
    pe = np.zeros((seq_len, model_dim), dtype=np.float32)
    pe[:, 0::2] = np.sin(positions / frequencies)
    pe[:, 1::2] = np.cos(positions / frequencies)
    return pe


_PE = _pe_table_np(MAX_SEQ_LEN, MODEL_DIM)

_NUM_WORKERS = 32
_P = 4
_NBUF = 3


def _make_sc_kernel(batch: int, seq_len: int, d: int):
    mesh = plsc.VectorSubcoreMesh(core_axis_name="c", subcore_axis_name="s")
    pos_per_w = seq_len // _NUM_WORKERS
    n_chunks = pos_per_w // _P
    n_rows = batch * seq_len
    dv = d // 16

    @functools.partial(
        pl.kernel,
        mesh=mesh,
        out_type=jax.ShapeDtypeStruct((n_rows, d), jnp.float32),
        scratch_types=[
            pltpu.VMEM((batch, pos_per_w), jnp.int32),
            pltpu.VMEM((_NBUF, batch, _P, d), jnp.float32),
            pltpu.VMEM((_NBUF, _P, d), jnp.float32),
            pltpu.SemaphoreType.DMA,
            pltpu.SemaphoreType.DMA,
            pltpu.SemaphoreType.DMA,
            pltpu.SemaphoreType.DMA,
            pltpu.SemaphoreType.DMA,
            pltpu.SemaphoreType.DMA,
        ],
    )
    def body(ids_hbm, table_hbm, pe_hbm, out_hbm, idx_v, rows_v, pe_v,
             gsem0, gsem1, gsem2, ssem0, ssem1, ssem2):
        nc = 2
        wid = lax.axis_index("s") * nc + lax.axis_index("c")
        pos0 = wid * pos_per_w

        pltpu.sync_copy(ids_hbm.at[pl.ds(wid * batch, batch)], idx_v)

        gsems = (gsem0, gsem1, gsem2)
        ssems = (ssem0, ssem1, ssem2)

        def load_descs(c, buf):
            descs = [
                pltpu.make_async_copy(
                    table_hbm.at[idx_v.at[b, pl.ds(c * _P, _P)]],
                    rows_v.at[buf, b], gsems[buf])
                for b in range(batch)
            ]
            descs.append(pltpu.make_async_copy(
                pe_hbm.at[pl.ds(pos0 + c * _P, _P)], pe_v.at[buf], gsems[buf]))
            return descs

        def store_descs(c, buf):
            return [
                pltpu.make_async_copy(
                    rows_v.at[buf, b],
                    out_hbm.at[pl.ds(b * seq_len + pos0 + c * _P, _P)],
                    ssems[buf])
                for b in range(batch)
            ]

        def multiply(buf):
            def mul_body(j, _):
                col = j * 16
                for p in range(_P):
                    pe_vec = pe_v[buf, p, pl.ds(col, 16)]
                    for b in range(batch):
                        rows_v[buf, b, p, pl.ds(col, 16)] = (
                            rows_v[buf, b, p, pl.ds(col, 16)] * pe_vec
                        )
                return 0

            lax.fori_loop(0, dv, mul_body, 0)

        for desc in load_descs(0, 0):
            desc.start()

        def chunk_step(c, buf):
            ahead = (buf + 1) % _NBUF

            @pl.when(c + 1 < n_chunks)
            def _():
                @pl.when(c >= 2)
                def _():
                    for desc in store_descs(c - 2, ahead):
                        desc.wait()

                for desc in load_descs(c + 1, ahead):
                    desc.start()

            for desc in load_descs(c, buf):
                desc.wait()
            multiply(buf)
            for desc in store_descs(c, buf):
                desc.start()

        def group_body(g, _):
            for local in range(_NBUF):
                c = _NBUF * g + local
                chunk_step(c, local)
            return 0

        n_groups = n_chunks // _NBUF
        lax.fori_loop(0, n_groups, group_body, 0)
        for c in range(n_groups * _NBUF, n_chunks):
            chunk_step(c, c % _NBUF)

        for c in range(n_chunks - _NBUF, n_chunks):
            for desc in store_descs(c, c % _NBUF):
                desc.wait()

    return body


def kernel(input_ids, embedding_table):
    b, seq_len = input_ids.shape
    d = embedding_table.shape[1]
    pos_per_w = seq_len // _NUM_WORKERS
    ids_perm = (
        input_ids.astype(jnp.int32)
        .reshape(b, _NUM_WORKERS, pos_per_w)
        .transpose(1, 0, 2)
        .reshape(_NUM_WORKERS * b, pos_per_w)
    )
    pe = jnp.asarray(_PE[:seq_len])
    out = _make_sc_kernel(b, seq_len, d)(ids_perm, embedding_table, pe)
    return out.reshape(b, seq_len, d)
